# baseline (speedup 1.0000x reference)
import jax
import jax.numpy as jnp
from jax import lax
from jax.experimental import pallas as pl
from jax.experimental.pallas import tpu as pltpu
from jax.experimental.pallas import tpu_sc as plsc

KMER = 1024
B, L = 4096, 200
LP = 205
NW = 32
QB = B // NW
CHL = 40
NCHL = L // CHL


def _sc_body(idx_hbm, mask_hbm, w4_hbm, t5_hbm,
             rates_hbm, csp_hbm,
             idx_v, mask_v, wt_v0, wt_v1, rates_v0, rates_v1,
             t5_tab,
             sin0, sin1, scsp0, scsp1, srat0, srat1):
    bt = lax.axis_index("s") * 2 + lax.axis_index("c")
    pltpu.sync_copy(t5_hbm, t5_tab)
    pltpu.sync_copy(idx_hbm.at[pl.ds(bt * QB, QB), :], idx_v)
    pltpu.sync_copy(mask_hbm.at[pl.ds(bt * QB, QB), :], mask_v)

    wts = [wt_v0, wt_v1]
    rvs = [rates_v0, rates_v1]
    sins = [sin0, sin1]
    scsps = [scsp0, scsp1]
    srats = [srat0, srat1]

    iota = lax.iota(jnp.int32, 16)
    rows = [iota + 16 * k for k in range(QB // 16)]

    def in_copy(c, b):
        return pltpu.async_copy(
            w4_hbm.at[pl.ds(c * CHL, CHL), bt], wts[b], sins[b])

    def out_copies(c, b):
        return (pltpu.async_copy(
                    wts[b], csp_hbm.at[pl.ds(c * CHL, CHL), bt], scsps[b]),
                pltpu.async_copy(
                    rvs[b], rates_hbm.at[pl.ds(c * CHL // 8, CHL // 8), bt],
                    srats[b]))

    in_h = {0: in_copy(0, 0)}
    out_h = {}
    for c in range(NCHL):
        b = c % 2
        if c + 1 < NCHL:
            if c >= 1:
                for h in out_h.pop(c - 1):
                    h.wait()
            in_h[c + 1] = in_copy(c + 1, 1 - b)
        in_h.pop(c).wait()

        wt_v = wts[b]
        rates_v = rvs[b]

        @plsc.parallel_loop(0, CHL, unroll=2)
        def body(l_loc):
            lt = l_loc >> 3
            s = l_loc & 7
            lvec = jnp.full((16,), l_loc + c * CHL, jnp.int32)
            for k in range(QB // 16):
                idx = plsc.load_gather(idx_v, [rows[k], lvec])
                m = plsc.load_gather(mask_v, [rows[k], lvec])
                idx5 = idx * 5
                r = plsc.load_gather(t5_tab, [idx5 + 4])
                rates_v[lt, s, pl.ds(16 * k, 16)] = jnp.exp(r * m)
                for cc in range(4):
                    s_c = plsc.load_gather(t5_tab, [idx5 + cc])
                    plsc.addupdate(
                        wt_v.at[l_loc, cc, pl.ds(16 * k, 16)], s_c * m)

        out_h[c] = out_copies(c, b)

    for c in (NCHL - 2, NCHL - 1):
        for h in out_h.pop(c, ()):
            h.wait()


@jax.jit
def _run(idx2, mask2, w4, t5):
    mesh = plsc.VectorSubcoreMesh(core_axis_name="c", subcore_axis_name="s")
    return pl.kernel(
        _sc_body,
        out_type=[jax.ShapeDtypeStruct((L // 8, NW, 8, QB), jnp.float32),
                  jax.ShapeDtypeStruct((L, NW, 4, QB), jnp.float32)],
        mesh=mesh,
        compiler_params=pltpu.CompilerParams(needs_layout_passes=False),
        scratch_types=[
            pltpu.VMEM((QB, L), jnp.int32),
            pltpu.VMEM((QB, L), jnp.float32),
            pltpu.VMEM((CHL, 4, QB), jnp.float32),
            pltpu.VMEM((CHL, 4, QB), jnp.float32),
            pltpu.VMEM((CHL // 8, 8, QB), jnp.float32),
            pltpu.VMEM((CHL // 8, 8, QB), jnp.float32),
            pltpu.VMEM((KMER * 5,), jnp.float32),
            pltpu.SemaphoreType.DMA,
            pltpu.SemaphoreType.DMA,
            pltpu.SemaphoreType.DMA,
            pltpu.SemaphoreType.DMA,
            pltpu.SemaphoreType.DMA,
            pltpu.SemaphoreType.DMA,
        ],
    )(idx2, mask2, w4, t5)


def kernel(encoded_parents, masks, wt_base_modifier, r_table, s_table):
    idx2 = encoded_parents.astype(jnp.int32)
    w4 = wt_base_modifier.reshape(NW, QB, L, 4).transpose(2, 0, 3, 1)
    t5 = jnp.concatenate([s_table, r_table], axis=1).reshape(-1)
    rates5, csp4 = _run(idx2, masks, w4, t5)
    rates = rates5.transpose(0, 2, 1, 3).reshape(L, B).T
    csp = csp4.transpose(1, 3, 0, 2).reshape(B, L, 4)
    return rates, csp


# speedup vs baseline: 1.0678x; 1.0678x over previous
import jax
import jax.numpy as jnp
from jax import lax
from jax.experimental import pallas as pl
from jax.experimental.pallas import tpu as pltpu
from jax.experimental.pallas import tpu_sc as plsc

KMER = 1024
B, L = 4096, 200
LP = 205
NW = 32
QB = B // NW
CHL = 40
NCHL = L // CHL


def _sc_body(idx_hbm, mask_hbm, w4_hbm, t5_hbm,
             rates_hbm, csp_hbm,
             idx_v, mask_v, wt_v0, wt_v1, rates_v0, rates_v1,
             t5_tab,
             sin0, sin1, scsp0, scsp1, srat0, srat1):
    bt = lax.axis_index("s") * 2 + lax.axis_index("c")
    pltpu.sync_copy(t5_hbm, t5_tab)
    pltpu.sync_copy(idx_hbm.at[pl.ds(bt * QB, QB), :], idx_v)
    pltpu.sync_copy(mask_hbm.at[pl.ds(bt * QB, QB), :], mask_v)

    wts = [wt_v0, wt_v1]
    rvs = [rates_v0, rates_v1]
    sins = [sin0, sin1]
    scsps = [scsp0, scsp1]
    srats = [srat0, srat1]

    iota = lax.iota(jnp.int32, 16)
    rows = [iota + 16 * k for k in range(QB // 16)]

    def in_copy(c, b):
        return pltpu.async_copy(
            w4_hbm.at[pl.ds(c * CHL, CHL), bt], wts[b], sins[b])

    def out_copies(c, b):
        return (pltpu.async_copy(
                    wts[b], csp_hbm.at[pl.ds(c * CHL, CHL), bt], scsps[b]),
                pltpu.async_copy(
                    rvs[b], rates_hbm.at[pl.ds(c * CHL // 8, CHL // 8), bt],
                    srats[b]))

    in_h = {0: in_copy(0, 0)}
    out_h = {}
    for c in range(NCHL):
        b = c % 2
        if c + 1 < NCHL:
            if c >= 1:
                for h in out_h.pop(c - 1):
                    h.wait()
            in_h[c + 1] = in_copy(c + 1, 1 - b)
        in_h.pop(c).wait()

        wt_v = wts[b]
        rates_v = rvs[b]

        @plsc.parallel_loop(0, CHL * (QB // 16), unroll=4)
        def body(n):
            l_loc = n >> 3
            k16 = (n & 7) * 16
            lt = l_loc >> 3
            s = l_loc & 7
            lvec = jnp.full((16,), l_loc + c * CHL, jnp.int32)
            rowv = iota + k16
            idx = plsc.load_gather(idx_v, [rowv, lvec])
            m = plsc.load_gather(mask_v, [rowv, lvec])
            idx5 = idx * 5
            r = plsc.load_gather(t5_tab, [idx5 + 4])
            rates_v[lt, s, pl.ds(k16, 16)] = jnp.exp(r * m)
            for cc in range(4):
                s_c = plsc.load_gather(t5_tab, [idx5 + cc])
                plsc.addupdate(
                    wt_v.at[l_loc, cc, pl.ds(k16, 16)], s_c * m)

        out_h[c] = out_copies(c, b)

    for c in (NCHL - 2, NCHL - 1):
        for h in out_h.pop(c, ()):
            h.wait()


@jax.jit
def _run(idx2, mask2, w4, t5):
    mesh = plsc.VectorSubcoreMesh(core_axis_name="c", subcore_axis_name="s")
    return pl.kernel(
        _sc_body,
        out_type=[jax.ShapeDtypeStruct((L // 8, NW, 8, QB), jnp.float32),
                  jax.ShapeDtypeStruct((L, NW, 4, QB), jnp.float32)],
        mesh=mesh,
        compiler_params=pltpu.CompilerParams(needs_layout_passes=False),
        scratch_types=[
            pltpu.VMEM((QB, L), jnp.int32),
            pltpu.VMEM((QB, L), jnp.float32),
            pltpu.VMEM((CHL, 4, QB), jnp.float32),
            pltpu.VMEM((CHL, 4, QB), jnp.float32),
            pltpu.VMEM((CHL // 8, 8, QB), jnp.float32),
            pltpu.VMEM((CHL // 8, 8, QB), jnp.float32),
            pltpu.VMEM((KMER * 5,), jnp.float32),
            pltpu.SemaphoreType.DMA,
            pltpu.SemaphoreType.DMA,
            pltpu.SemaphoreType.DMA,
            pltpu.SemaphoreType.DMA,
            pltpu.SemaphoreType.DMA,
            pltpu.SemaphoreType.DMA,
        ],
    )(idx2, mask2, w4, t5)


def kernel(encoded_parents, masks, wt_base_modifier, r_table, s_table):
    idx2 = encoded_parents.astype(jnp.int32)
    w4 = wt_base_modifier.reshape(NW, QB, L, 4).transpose(2, 0, 3, 1)
    t5 = jnp.concatenate([s_table, r_table], axis=1).reshape(-1)
    rates5, csp4 = _run(idx2, masks, w4, t5)
    rates = rates5.transpose(0, 2, 1, 3).reshape(L, B).T
    csp = csp4.transpose(1, 3, 0, 2).reshape(B, L, 4)
    return rates, csp


# speedup vs baseline: 1.4590x; 1.3664x over previous
import jax
import jax.numpy as jnp
from jax import lax
from jax.experimental import pallas as pl
from jax.experimental.pallas import tpu as pltpu
from jax.experimental.pallas import tpu_sc as plsc

KMER = 1024
B, L = 4096, 200
LP = 205
NW = 32
QB = B // NW
CHL = 40
NCHL = L // CHL


def _sc_body(idx_hbm, mask_hbm, w4_hbm, t5_hbm,
             rates_hbm, csp_hbm,
             idx_p, mask_p, wt_v0, wt_v1, rates_v0, rates_v1,
             t5_tab,
             sin0, sin1, scsp0, scsp1, srat0, srat1):
    bt = lax.axis_index("s") * 2 + lax.axis_index("c")
    pltpu.sync_copy(t5_hbm, t5_tab)

    def stage(src_hbm, dst_p, tmp):
        for half in range(2):
            pltpu.sync_copy(
                src_hbm.at[pl.ds(bt * QB + half * (QB // 2), QB // 2), :],
                tmp)

            @plsc.parallel_loop(0, QB // 2, unroll=2)
            def rp(q):
                base = (half * (QB // 2) + q) * LP
                for j in range(13):
                    l0 = 16 * j if j < 12 else L - 16
                    dst_p[pl.ds(base + l0, 16)] = tmp[q, pl.ds(l0, 16)]

    pl.run_scoped(lambda tmp: stage(idx_hbm, idx_p, tmp),
                  pltpu.VMEM((QB // 2, L), jnp.int32))
    pl.run_scoped(lambda tmp: stage(mask_hbm, mask_p, tmp),
                  pltpu.VMEM((QB // 2, L), jnp.float32))

    wts = [wt_v0, wt_v1]
    rvs = [rates_v0, rates_v1]
    sins = [sin0, sin1]
    scsps = [scsp0, scsp1]
    srats = [srat0, srat1]

    iota205 = lax.iota(jnp.int32, 16) * LP

    def in_copy(c, b):
        return pltpu.async_copy(
            w4_hbm.at[pl.ds(c * CHL, CHL), bt], wts[b], sins[b])

    def out_copies(c, b):
        return (pltpu.async_copy(
                    wts[b], csp_hbm.at[pl.ds(c * CHL, CHL), bt], scsps[b]),
                pltpu.async_copy(
                    rvs[b], rates_hbm.at[pl.ds(c * CHL // 8, CHL // 8), bt],
                    srats[b]))

    in_h = {0: in_copy(0, 0)}
    out_h = {}
    for c in range(NCHL):
        b = c % 2
        if c + 1 < NCHL:
            if c >= 1:
                for h in out_h.pop(c - 1):
                    h.wait()
            in_h[c + 1] = in_copy(c + 1, 1 - b)
        in_h.pop(c).wait()

        wt_v = wts[b]
        rates_v = rvs[b]

        @plsc.parallel_loop(0, CHL * (QB // 16), unroll=4)
        def body(n):
            l_loc = n >> 3
            k16 = (n & 7) * 16
            lt = l_loc >> 3
            s = l_loc & 7
            pos = iota205 + (k16 * LP + l_loc + c * CHL)
            idx = plsc.load_gather(idx_p, [pos])
            m = plsc.load_gather(mask_p, [pos])
            idx5 = idx * 5
            r = plsc.load_gather(t5_tab, [idx5 + 4])
            rates_v[lt, s, pl.ds(k16, 16)] = jnp.exp(r * m)
            for cc in range(4):
                s_c = plsc.load_gather(t5_tab, [idx5 + cc])
                plsc.addupdate(
                    wt_v.at[l_loc, cc, pl.ds(k16, 16)], s_c * m)

        out_h[c] = out_copies(c, b)

    for c in (NCHL - 2, NCHL - 1):
        for h in out_h.pop(c, ()):
            h.wait()


@jax.jit
def _run(idx2, mask2, w4, t5):
    mesh = plsc.VectorSubcoreMesh(core_axis_name="c", subcore_axis_name="s")
    return pl.kernel(
        _sc_body,
        out_type=[jax.ShapeDtypeStruct((L // 8, NW, 8, QB), jnp.float32),
                  jax.ShapeDtypeStruct((L, NW, 4, QB), jnp.float32)],
        mesh=mesh,
        compiler_params=pltpu.CompilerParams(needs_layout_passes=False),
        scratch_types=[
            pltpu.VMEM((QB * LP,), jnp.int32),
            pltpu.VMEM((QB * LP,), jnp.float32),
            pltpu.VMEM((CHL, 4, QB), jnp.float32),
            pltpu.VMEM((CHL, 4, QB), jnp.float32),
            pltpu.VMEM((CHL // 8, 8, QB), jnp.float32),
            pltpu.VMEM((CHL // 8, 8, QB), jnp.float32),
            pltpu.VMEM((KMER * 5,), jnp.float32),
            pltpu.SemaphoreType.DMA,
            pltpu.SemaphoreType.DMA,
            pltpu.SemaphoreType.DMA,
            pltpu.SemaphoreType.DMA,
            pltpu.SemaphoreType.DMA,
            pltpu.SemaphoreType.DMA,
        ],
    )(idx2, mask2, w4, t5)


def kernel(encoded_parents, masks, wt_base_modifier, r_table, s_table):
    idx2 = encoded_parents.astype(jnp.int32)
    w4 = wt_base_modifier.reshape(NW, QB, L, 4).transpose(2, 0, 3, 1)
    t5 = jnp.concatenate([s_table, r_table], axis=1).reshape(-1)
    rates5, csp4 = _run(idx2, masks, w4, t5)
    rates = rates5.transpose(0, 2, 1, 3).reshape(L, B).T
    csp = csp4.transpose(1, 3, 0, 2).reshape(B, L, 4)
    return rates, csp


# speedup vs baseline: 1.4814x; 1.0154x over previous
import jax
import jax.numpy as jnp
from jax import lax
from jax.experimental import pallas as pl
from jax.experimental.pallas import tpu as pltpu
from jax.experimental.pallas import tpu_sc as plsc

KMER = 1024
B, L = 4096, 200
LP = 205
NW = 32
QB = B // NW
CHL = 40
NCHL = L // CHL


def _sc_body(idx_hbm, mask_hbm, w4_hbm, t5_hbm,
             rates_hbm, csp_hbm,
             idx_p, mask_p, wt_v0, wt_v1, rates_v0, rates_v1,
             t5_tab,
             sin0, sin1, scsp0, scsp1, srat0, srat1):
    bt = lax.axis_index("s") * 2 + lax.axis_index("c")

    wts = [wt_v0, wt_v1]
    rvs = [rates_v0, rates_v1]
    sins = [sin0, sin1]
    scsps = [scsp0, scsp1]
    srats = [srat0, srat1]

    iota205 = lax.iota(jnp.int32, 16) * LP

    def in_copy(c, b):
        return pltpu.async_copy(
            w4_hbm.at[pl.ds(c * CHL, CHL), bt], wts[b], sins[b])

    def out_copies(c, b):
        return (pltpu.async_copy(
                    wts[b], csp_hbm.at[pl.ds(c * CHL, CHL), bt], scsps[b]),
                pltpu.async_copy(
                    rvs[b], rates_hbm.at[pl.ds(c * CHL // 8, CHL // 8), bt],
                    srats[b]))

    in_h = {0: in_copy(0, 0), 1: in_copy(1, 1)}
    pltpu.sync_copy(t5_hbm, t5_tab)

    QQ = QB // 4

    def stage(tmp_i, tmp_f):
        for q in range(4):
            hi = pltpu.async_copy(
                idx_hbm.at[pl.ds(bt * QB + q * QQ, QQ), :], tmp_i, scsp0)
            hm = pltpu.async_copy(
                mask_hbm.at[pl.ds(bt * QB + q * QQ, QQ), :], tmp_f, scsp1)
            hi.wait()
            hm.wait()

            @plsc.parallel_loop(0, QQ, unroll=2)
            def rp(r):
                base = (q * QQ + r) * LP
                for j in range(13):
                    l0 = 16 * j if j < 12 else L - 16
                    idx_p[pl.ds(base + l0, 16)] = tmp_i[r, pl.ds(l0, 16)]
                    mask_p[pl.ds(base + l0, 16)] = tmp_f[r, pl.ds(l0, 16)]

    pl.run_scoped(stage,
                  pltpu.VMEM((QB // 4, L), jnp.int32),
                  pltpu.VMEM((QB // 4, L), jnp.float32))

    out_h = {}
    for c in range(NCHL):
        b = c % 2
        if c + 1 < NCHL:
            if c >= 1:
                for h in out_h.pop(c - 1):
                    h.wait()
            in_h[c + 1] = in_copy(c + 1, 1 - b)
        in_h.pop(c).wait()

        wt_v = wts[b]
        rates_v = rvs[b]

        @plsc.parallel_loop(0, CHL * (QB // 16), unroll=4)
        def body(n):
            l_loc = n >> 3
            k16 = (n & 7) * 16
            lt = l_loc >> 3
            s = l_loc & 7
            pos = iota205 + (k16 * LP + l_loc + c * CHL)
            idx = plsc.load_gather(idx_p, [pos])
            m = plsc.load_gather(mask_p, [pos])
            idx5 = idx * 5
            r = plsc.load_gather(t5_tab, [idx5 + 4])
            rates_v[lt, s, pl.ds(k16, 16)] = jnp.exp(r * m)
            for cc in range(4):
                s_c = plsc.load_gather(t5_tab, [idx5 + cc])
                plsc.addupdate(
                    wt_v.at[l_loc, cc, pl.ds(k16, 16)], s_c * m)

        out_h[c] = out_copies(c, b)

    for c in (NCHL - 2, NCHL - 1):
        for h in out_h.pop(c, ()):
            h.wait()


@jax.jit
def _run(idx2, mask2, w4, t5):
    mesh = plsc.VectorSubcoreMesh(core_axis_name="c", subcore_axis_name="s")
    return pl.kernel(
        _sc_body,
        out_type=[jax.ShapeDtypeStruct((L // 8, NW, 8, QB), jnp.float32),
                  jax.ShapeDtypeStruct((L, NW, 4, QB), jnp.float32)],
        mesh=mesh,
        compiler_params=pltpu.CompilerParams(needs_layout_passes=False),
        scratch_types=[
            pltpu.VMEM((QB * LP,), jnp.int32),
            pltpu.VMEM((QB * LP,), jnp.float32),
            pltpu.VMEM((CHL, 4, QB), jnp.float32),
            pltpu.VMEM((CHL, 4, QB), jnp.float32),
            pltpu.VMEM((CHL // 8, 8, QB), jnp.float32),
            pltpu.VMEM((CHL // 8, 8, QB), jnp.float32),
            pltpu.VMEM((KMER * 5,), jnp.float32),
            pltpu.SemaphoreType.DMA,
            pltpu.SemaphoreType.DMA,
            pltpu.SemaphoreType.DMA,
            pltpu.SemaphoreType.DMA,
            pltpu.SemaphoreType.DMA,
            pltpu.SemaphoreType.DMA,
        ],
    )(idx2, mask2, w4, t5)


def kernel(encoded_parents, masks, wt_base_modifier, r_table, s_table):
    idx2 = encoded_parents.astype(jnp.int32)
    w4 = wt_base_modifier.reshape(NW, QB, L, 4).transpose(2, 0, 3, 1)
    t5 = jnp.concatenate([s_table, r_table], axis=1).reshape(-1)
    rates5, csp4 = _run(idx2, masks, w4, t5)
    rates = rates5.transpose(0, 2, 1, 3).reshape(L, B).T
    csp = csp4.transpose(1, 3, 0, 2).reshape(B, L, 4)
    return rates, csp
